# Initial kernel scaffold; baseline (speedup 1.0000x reference)
import jax
import jax.numpy as jnp
from jax.experimental import pallas as pl


def kernel(x, edge_index, edge_attr, W_node, b_node, W_edge, b_edge, W_layers, b_layers, W_dec, b_dec):
    raise NotImplementedError("write your pallas kernel here")


# speedup vs baseline: 3.1119x; 3.1119x over previous
import functools

import jax
import jax.numpy as jnp
from jax import lax
from jax.experimental import pallas as pl
from jax.experimental.pallas import tpu as pltpu
from jax.experimental.pallas import tpu_sc as plsc

NC = 2
NS = 16
LANES = 16
NUM_GRAPHS = 10

F32 = jnp.float32


def _round_up(v, m):
    return (v + m - 1) // m * m


def _sc_gather_segsum(nf2, src, dst, n, e, w, ch=80, zr=128):
    npad = _round_up(n, NS * zr)
    rpt = npad // NS
    ept = e // NS
    assert e % NS == 0 and ept % ch == 0 and rpt % zr == 0
    mesh = plsc.VectorSubcoreMesh(core_axis_name="c", subcore_axis_name="s")

    def body(nf2_hbm, src_hbm, dst_hbm, out_hbm, acc, idxs, idxd, rows, zbuf, sem):
        c = lax.axis_index("c")
        s = lax.axis_index("s")
        zero = jnp.zeros((LANES,), F32)

        def _zb(i, carry):
            for j in range(w // LANES):
                zbuf[i, pl.ds(j * LANES, LANES)] = zero
            return carry

        lax.fori_loop(0, zr, _zb, 0)
        for k in range(rpt // zr):
            pltpu.sync_copy(zbuf, acc.at[pl.ds(s * rpt + k * zr, zr)])
        plsc.subcore_barrier()

        base0 = s * ept
        coff = c * n
        ooff = c * npad

        def _chunk(j, carry):
            base = pl.multiple_of(base0 + j * ch, 16)
            pltpu.sync_copy(src_hbm.at[pl.ds(base, ch)], idxs)
            for k in range(ch // LANES):
                sl = pl.ds(k * LANES, LANES)
                idxs[sl] = idxs[sl] + coff
            pltpu.async_copy(nf2_hbm.at[idxs], rows, sem).wait()
            pltpu.sync_copy(dst_hbm.at[pl.ds(base, ch)], idxd)
            pltpu.sync_copy(rows, acc.at[idxd], add=True)
            return carry

        lax.fori_loop(0, ept // ch, _chunk, 0)
        plsc.subcore_barrier()
        pltpu.sync_copy(acc.at[pl.ds(s * rpt, rpt)],
                        out_hbm.at[pl.ds(ooff + s * rpt, rpt)])

    call = pl.kernel(
        body,
        out_type=jax.ShapeDtypeStruct((2 * npad, w), F32),
        mesh=mesh,
        scratch_types=[
            pltpu.VMEM_SHARED((npad, w), F32),
            pltpu.VMEM((ch,), jnp.int32),
            pltpu.VMEM((ch,), jnp.int32),
            pltpu.VMEM((ch, w), F32),
            pltpu.VMEM((zr, w), F32),
            pltpu.SemaphoreType.DMA,
        ],
    )
    return call(nf2, src, dst)


def _sc_edge_segsum(ea, dst, n, e, w=128, ch=80, zr=128):
    npad = _round_up(n, NS * zr)
    rpt = npad // NS
    ept = e // (NC * NS)
    assert e % (NC * NS) == 0 and ept % ch == 0 and rpt % zr == 0
    mesh = plsc.VectorSubcoreMesh(core_axis_name="c", subcore_axis_name="s")

    def body(ea_hbm, dst_hbm, out_hbm, acc, idxd, rows, zbuf):
        c = lax.axis_index("c")
        s = lax.axis_index("s")
        zero = jnp.zeros((LANES,), F32)

        def _zb(i, carry):
            for j in range(w // LANES):
                zbuf[i, pl.ds(j * LANES, LANES)] = zero
            return carry

        lax.fori_loop(0, zr, _zb, 0)
        for k in range(rpt // zr):
            pltpu.sync_copy(zbuf, acc.at[pl.ds(s * rpt + k * zr, zr)])
        plsc.subcore_barrier()

        base0 = (c * NS + s) * ept

        def _chunk(j, carry):
            base = pl.multiple_of(base0 + j * ch, 16)
            pltpu.sync_copy(ea_hbm.at[pl.ds(base, ch)], rows)
            pltpu.sync_copy(dst_hbm.at[pl.ds(base, ch)], idxd)
            pltpu.sync_copy(rows, acc.at[idxd], add=True)
            return carry

        lax.fori_loop(0, ept // ch, _chunk, 0)
        plsc.subcore_barrier()
        pltpu.sync_copy(acc.at[pl.ds(s * rpt, rpt)],
                        out_hbm.at[pl.ds(c * npad + s * rpt, rpt)])

    call = pl.kernel(
        body,
        out_type=jax.ShapeDtypeStruct((2 * npad, w), F32),
        mesh=mesh,
        scratch_types=[
            pltpu.VMEM_SHARED((npad, w), F32),
            pltpu.VMEM((ch,), jnp.int32),
            pltpu.VMEM((ch, w), F32),
            pltpu.VMEM((zr, w), F32),
        ],
    )
    return call(ea, dst)


def _tc_encode(x, w, b, n, d_in, d, r=1000):
    h = d // 2

    def body(x_ref, w_ref, b_ref, o_ref):
        o_ref[0] = (jnp.dot(x_ref[...], w_ref[...], preferred_element_type=F32)
                    + b_ref[...])

    return pl.pallas_call(
        body,
        grid=(2, n // r),
        in_specs=[
            pl.BlockSpec((r, d_in), lambda i, j: (j, 0)),
            pl.BlockSpec((d_in, h), lambda i, j: (0, i)),
            pl.BlockSpec((1, h), lambda i, j: (0, i)),
        ],
        out_specs=pl.BlockSpec((1, r, h), lambda i, j: (i, j, 0)),
        out_shape=jax.ShapeDtypeStruct((2, n, h), F32),
    )(x, w, b)


def _tc_cagg(raw2, w_edge, b_edge, n, d_e, d, r=1000):
    h = d // 2
    wa = raw2.shape[-1]

    def body(r0, r1, w_ref, b_ref, o_ref):
        ea = r0[0, :, :d_e] + r1[0, :, :d_e]
        deg = r0[0, :, d_e:d_e + 1] + r1[0, :, d_e:d_e + 1]
        o_ref[...] = (jnp.dot(ea, w_ref[...], preferred_element_type=F32)
                      + deg * b_ref[...])

    return pl.pallas_call(
        body,
        grid=(2, n // r),
        in_specs=[
            pl.BlockSpec((1, r, wa), lambda i, j: (0, j, 0)),
            pl.BlockSpec((1, r, wa), lambda i, j: (1, j, 0)),
            pl.BlockSpec((d_e, h), lambda i, j: (0, i)),
            pl.BlockSpec((1, h), lambda i, j: (0, i)),
        ],
        out_specs=pl.BlockSpec((r, h), lambda i, j: (j, i)),
        out_shape=jax.ShapeDtypeStruct((n, d), F32),
    )(raw2, raw2, w_edge, b_edge)


def _tc_update(s2, cagg, nf2, w, b, n, d, r=1000):
    h = d // 2

    def body(slo, shi, cg, nf_ref, w_ref, b_ref, o_ref):
        full = jnp.concatenate([slo[0], shi[0]], axis=-1) + cg[...]
        z = jnp.dot(full, w_ref[...], preferred_element_type=F32) + b_ref[...]
        o_ref[0] = jnp.maximum(z, 0.0) + nf_ref[0]

    return pl.pallas_call(
        body,
        grid=(2, n // r),
        in_specs=[
            pl.BlockSpec((1, r, h), lambda i, j: (0, j, 0)),
            pl.BlockSpec((1, r, h), lambda i, j: (1, j, 0)),
            pl.BlockSpec((r, d), lambda i, j: (j, 0)),
            pl.BlockSpec((1, r, h), lambda i, j: (i, j, 0)),
            pl.BlockSpec((d, h), lambda i, j: (0, i)),
            pl.BlockSpec((1, h), lambda i, j: (0, i)),
        ],
        out_specs=pl.BlockSpec((1, r, h), lambda i, j: (i, j, 0)),
        out_shape=jax.ShapeDtypeStruct((2, n, h), F32),
    )(s2, s2, cagg, nf2, w, b)


def _tc_decode(nf2, w_dec_row, b_dec, n, d, r=1000):
    h = d // 2

    def body(lo, hi, w_ref, b_ref, o_ref):
        full = jnp.concatenate([lo[0], hi[0]], axis=-1)
        o_ref[...] = jnp.sum(full * w_ref[...], axis=1, keepdims=True) + b_ref[...]

    return pl.pallas_call(
        body,
        grid=(n // r,),
        in_specs=[
            pl.BlockSpec((1, r, h), lambda j: (0, j, 0)),
            pl.BlockSpec((1, r, h), lambda j: (1, j, 0)),
            pl.BlockSpec((1, d), lambda j: (0, 0)),
            pl.BlockSpec((1, 1), lambda j: (0, 0)),
        ],
        out_specs=pl.BlockSpec((r, 1), lambda j: (j, 0)),
        out_shape=jax.ShapeDtypeStruct((n, 1), F32),
    )(nf2, nf2, w_dec_row, b_dec)


def kernel(x, edge_index, edge_attr, W_node, b_node, W_edge, b_edge,
           W_layers, b_layers, W_dec, b_dec):
    n, d_in = x.shape
    e = edge_index.shape[1]
    d = W_node.shape[1]
    d_e = W_edge.shape[0]
    num_layers = W_layers.shape[0]
    h = d // 2

    src = edge_index[0]
    dst = edge_index[1]

    ea_aug = jnp.concatenate(
        [edge_attr,
         jnp.ones((e, 1), F32),
         jnp.zeros((e, 128 - d_e - 1), F32)], axis=1)
    npad = _round_up(n, NS * 128)
    raw = _sc_edge_segsum(ea_aug, dst, n, e, w=128)
    cagg = _tc_cagg(raw.reshape(2, npad, 128), W_edge, b_edge.reshape(1, d),
                    n, d_e, d)

    nf2 = _tc_encode(x, W_node, b_node.reshape(1, d), n, d_in, d)
    for l in range(num_layers):
        s = _sc_gather_segsum(nf2.reshape(2 * n, h), src, dst, n, e, h)
        nf2 = _tc_update(s.reshape(2, npad, h), cagg, nf2,
                         W_layers[l], b_layers[l].reshape(1, d), n, d)

    out = _tc_decode(nf2, W_dec.reshape(1, d), b_dec.reshape(1, 1), n, d)
    return out.reshape(NUM_GRAPHS, -1, 1)
